# baseline (speedup 1.0000x reference)
import functools

import jax
import jax.numpy as jnp
from jax import lax
from jax.experimental import pallas as pl
from jax.experimental.pallas import tpu as pltpu
from jax.experimental.pallas import tpu_sc as plsc

_B = 16384
_L = 200
_D = 4
_NC = 2
_NS = 16
_NW = _NC * _NS
_BG = 4
_LG = _NW // _BG
_BC = _B // _BG
_LPW = _L // _LG
_BT = _BC // 128
_RPC = 1
_NCH = _LPW // _RPC


def _sc_lookup(idx2, pk):
  mesh = plsc.VectorSubcoreMesh(core_axis_name="c", subcore_axis_name="s")

  @functools.partial(
      pl.kernel,
      mesh=mesh,
      compiler_params=pltpu.CompilerParams(needs_layout_passes=False),
      out_type=jax.ShapeDtypeStruct((_L * _D * (_B // 128), 128), jnp.float32),
      scratch_types=[
          pltpu.VMEM((32,), jnp.float32),
          pltpu.VMEM((_RPC, _BC), jnp.int32),
          pltpu.VMEM((_RPC, _BC), jnp.int32),
          pltpu.VMEM((_RPC, _BC), jnp.int32),
          pltpu.VMEM((_RPC, _BC), jnp.int32),
          pltpu.VMEM((_BT * _D, 128), jnp.float32),
          pltpu.VMEM((_BT * _D, 128), jnp.float32),
          pltpu.VMEM((_BT * _D, 128), jnp.float32),
          pltpu.VMEM((_BT * _D, 128), jnp.float32),
          pltpu.SemaphoreType.DMA,
          pltpu.SemaphoreType.DMA,
          pltpu.SemaphoreType.DMA,
          pltpu.SemaphoreType.DMA,
          pltpu.SemaphoreType.DMA,
          pltpu.SemaphoreType.DMA,
          pltpu.SemaphoreType.DMA,
          pltpu.SemaphoreType.DMA,
      ],
  )
  def k(idx_hbm, pk_hbm, out_hbm, pk_v, idx_v0, idx_v1, idx_v2, idx_v3,
        out_v0, out_v1, out_v2, out_v3,
        si0, si1, si2, si3, so0, so1, so2, so3):
    wid = lax.axis_index("s") * _NC + lax.axis_index("c")
    lgrp = wid // _BG
    bgrp = wid % _BG
    b0 = bgrp * _BC
    pltpu.sync_copy(pk_hbm, pk_v)
    def splat(i):
      return plsc.load_gather(pk_v, [jnp.full((16,), i, jnp.int32)])

    lut = [[splat(2 + 2 * v) * splat(8 + c) + splat(3 + 2 * v) * splat(12 + c)
            + splat(16 + c) for c in range(_D)] for v in range(3)]

    idx_bufs = (idx_v0, idx_v1, idx_v2, idx_v3)
    out_bufs = (out_v0, out_v1, out_v2, out_v3)
    in_sems, out_sems = (si0, si1, si2, si3), (so0, so1, so2, so3)

    def idx_dma(ch, p):
      l = lgrp * _LPW + ch * _RPC
      return pltpu.async_copy(
          idx_hbm.at[pl.ds(l, _RPC), pl.ds(b0, _BC)], idx_bufs[p],
          in_sems[p])

    def out_dma(kk, q):
      l = lgrp * _LPW + kk
      return pltpu.async_copy(
          out_bufs[q],
          out_hbm.at[pl.ds(l * _D * (_B // 128) + bgrp * _BT * _D,
                           _BT * _D), :],
          out_sems[q])

    in_h = [None, None, None, None]
    for ch in range(min(3, _NCH)):
      in_h[ch & 3] = idx_dma(ch, ch & 3)
    out_h = [None, None, None, None]
    for ch in range(_NCH):
      p = ch & 3
      if ch + 3 < _NCH:
        in_h[(ch + 3) & 3] = idx_dma(ch + 3, (ch + 3) & 3)
      in_h[p].wait()
      idx_v = idx_bufs[p]
      for j in range(_RPC):
        kk = ch * _RPC + j
        q = kk & 3
        if out_h[q] is not None:
          out_h[q].wait()
        out_v = out_bufs[q]

        def body(bt, carry):
          for m in range(8):
            iv = idx_v[j, pl.ds(bt * 128 + m * 16, 16)]
            m0 = iv == 0
            m1 = iv == 1
            for c in range(_D):
              out_v[bt * _D + c, pl.ds(m * 16, 16)] = jnp.where(
                  m0, lut[0][c], jnp.where(m1, lut[1][c], lut[2][c]))
          return carry

        lax.fori_loop(0, _BT, body, 0)
        out_h[q] = out_dma(kk, q)
    for q in range(4):
      if out_h[q] is not None:
        out_h[q].wait()

  return k(idx2, pk)


def kernel(inputs, table, W, b):
  idx2 = inputs.T.astype(jnp.int32)
  pk = jnp.zeros((32,), jnp.float32)
  pk = pk.at[2:8].set(table.reshape(-1).astype(jnp.float32))
  pk = pk.at[8:16].set(W.reshape(-1).astype(jnp.float32))
  pk = pk.at[16:20].set(b.astype(jnp.float32))
  out = _sc_lookup(idx2, pk)
  out = out.reshape(_L, _B // 128, _D, 128)
  return out.transpose(1, 3, 0, 2).reshape(_B, _L, _D)


# speedup vs baseline: 1.0043x; 1.0043x over previous
import functools

import jax
import jax.numpy as jnp
from jax import lax
from jax.experimental import pallas as pl
from jax.experimental.pallas import tpu as pltpu
from jax.experimental.pallas import tpu_sc as plsc

_B = 16384
_L = 200
_D = 4
_NC = 2
_NS = 16
_NW = _NC * _NS
_BC = _B // _NW
_RB = _L // 8
_BT = _BC // 128


def _sc_lookup(idx2, pk):
  mesh = plsc.VectorSubcoreMesh(core_axis_name="c", subcore_axis_name="s")

  @functools.partial(
      pl.kernel,
      mesh=mesh,
      compiler_params=pltpu.CompilerParams(needs_layout_passes=False),
      out_type=jax.ShapeDtypeStruct((_L, _B // 128 * _D, 128), jnp.float32),
      scratch_types=[
          pltpu.VMEM((32,), jnp.float32),
          pltpu.VMEM((8, _BC), jnp.int32),
          pltpu.VMEM((8, _BC), jnp.int32),
          pltpu.VMEM((8, _BC), jnp.int32),
          pltpu.VMEM((8, _BC), jnp.int32),
          pltpu.VMEM((8, _BT * _D, 128), jnp.float32),
          pltpu.VMEM((8, _BT * _D, 128), jnp.float32),
          pltpu.VMEM((8, _BT * _D, 128), jnp.float32),
          pltpu.VMEM((8, _BT * _D, 128), jnp.float32),
          pltpu.SemaphoreType.DMA,
          pltpu.SemaphoreType.DMA,
          pltpu.SemaphoreType.DMA,
          pltpu.SemaphoreType.DMA,
          pltpu.SemaphoreType.DMA,
          pltpu.SemaphoreType.DMA,
          pltpu.SemaphoreType.DMA,
          pltpu.SemaphoreType.DMA,
      ],
  )
  def k(idx_hbm, pk_hbm, out_hbm, pk_v, idx_v0, idx_v1, idx_v2, idx_v3,
        out_v0, out_v1, out_v2, out_v3,
        si0, si1, si2, si3, so0, so1, so2, so3):
    wid = lax.axis_index("s") * _NC + lax.axis_index("c")
    b0 = wid * _BC
    pltpu.sync_copy(pk_hbm, pk_v)
    def splat(i):
      return plsc.load_gather(pk_v, [jnp.full((16,), i, jnp.int32)])

    lut = [[splat(2 + 2 * v) * splat(8 + c) + splat(3 + 2 * v) * splat(12 + c)
            + splat(16 + c) for c in range(_D)] for v in range(3)]

    idx_bufs = (idx_v0, idx_v1, idx_v2, idx_v3)
    out_bufs = (out_v0, out_v1, out_v2, out_v3)
    in_sems, out_sems = (si0, si1, si2, si3), (so0, so1, so2, so3)

    def idx_dma(rb, p):
      return pltpu.async_copy(
          idx_hbm.at[pl.ds(rb * 8, 8), pl.ds(b0, _BC)], idx_bufs[p],
          in_sems[p])

    def out_dma(rb, p):
      return pltpu.async_copy(
          out_bufs[p],
          out_hbm.at[pl.ds(rb * 8, 8), pl.ds(wid * _BT * _D, _BT * _D), :],
          out_sems[p])

    in_h = [None, None, None, None]
    for rb in range(min(3, _RB)):
      in_h[rb & 3] = idx_dma(rb, rb & 3)
    out_h = [None, None, None, None]
    for rb in range(_RB):
      p = rb & 3
      if rb + 3 < _RB:
        in_h[(rb + 3) & 3] = idx_dma(rb + 3, (rb + 3) & 3)
      in_h[p].wait()
      if out_h[p] is not None:
        out_h[p].wait()
      idx_v, out_v = idx_bufs[p], out_bufs[p]

      def body(g, carry):
        j = g >> 2
        bt = g & 3
        for m in range(8):
          iv = idx_v[j, pl.ds(bt * 128 + m * 16, 16)]
          m0 = iv == 0
          m1 = iv == 1
          for c in range(_D):
            out_v[j, bt * _D + c, pl.ds(m * 16, 16)] = jnp.where(
                m0, lut[0][c], jnp.where(m1, lut[1][c], lut[2][c]))
        return carry

      lax.fori_loop(0, 8 * _BT, body, 0)
      out_h[p] = out_dma(rb, p)
    for p in range(4):
      if out_h[p] is not None:
        out_h[p].wait()

  return k(idx2, pk)


def kernel(inputs, table, W, b):
  idx2 = inputs.T.astype(jnp.int32)
  pk = jnp.zeros((32,), jnp.float32)
  pk = pk.at[2:8].set(table.reshape(-1).astype(jnp.float32))
  pk = pk.at[8:16].set(W.reshape(-1).astype(jnp.float32))
  pk = pk.at[16:20].set(b.astype(jnp.float32))
  out = _sc_lookup(idx2, pk)
  out = out.reshape(_L, _B // 128, _D, 128)
  return out.transpose(1, 3, 0, 2).reshape(_B, _L, _D)


# speedup vs baseline: 1.4107x; 1.4047x over previous
import functools

import jax
import jax.numpy as jnp
from jax import lax
from jax.experimental import pallas as pl
from jax.experimental.pallas import tpu as pltpu
from jax.experimental.pallas import tpu_sc as plsc

_B = 16384
_L = 200
_D = 4
_NC = 2
_NS = 16
_NW = _NC * _NS
_BC = _B // _NW
_RB = _L // 8
_BT = _BC // 128


def _sc_lookup(idx2, pk):
  mesh = plsc.VectorSubcoreMesh(core_axis_name="c", subcore_axis_name="s")

  @functools.partial(
      pl.kernel,
      mesh=mesh,
      compiler_params=pltpu.CompilerParams(needs_layout_passes=False),
      out_type=jax.ShapeDtypeStruct((_L, _B // 128 * _D, 128), jnp.float32),
      scratch_types=[
          pltpu.VMEM((32,), jnp.float32),
          pltpu.VMEM((8, _BC), jnp.int32),
          pltpu.VMEM((8, _BC), jnp.int32),
          pltpu.VMEM((8, _BC), jnp.int32),
          pltpu.VMEM((8, _BC), jnp.int32),
          pltpu.VMEM((8, _BT * _D, 128), jnp.float32),
          pltpu.VMEM((8, _BT * _D, 128), jnp.float32),
          pltpu.VMEM((8, _BT * _D, 128), jnp.float32),
          pltpu.VMEM((8, _BT * _D, 128), jnp.float32),
          pltpu.SemaphoreType.DMA,
          pltpu.SemaphoreType.DMA,
          pltpu.SemaphoreType.DMA,
          pltpu.SemaphoreType.DMA,
          pltpu.SemaphoreType.DMA,
          pltpu.SemaphoreType.DMA,
          pltpu.SemaphoreType.DMA,
          pltpu.SemaphoreType.DMA,
      ],
  )
  def k(idx_hbm, pk_hbm, out_hbm, pk_v, idx_v0, idx_v1, idx_v2, idx_v3,
        out_v0, out_v1, out_v2, out_v3,
        si0, si1, si2, si3, so0, so1, so2, so3):
    wid = lax.axis_index("s") * _NC + lax.axis_index("c")
    b0 = wid * _BC
    pltpu.sync_copy(pk_hbm, pk_v)
    def splat(i):
      return plsc.load_gather(pk_v, [jnp.full((16,), i, jnp.int32)])

    lut = [[splat(2 + 2 * v) * splat(8 + c) + splat(3 + 2 * v) * splat(12 + c)
            + splat(16 + c) for c in range(_D)] for v in range(3)]

    idx_bufs = (idx_v0, idx_v1, idx_v2, idx_v3)
    out_bufs = (out_v0, out_v1, out_v2, out_v3)
    in_sems, out_sems = (si0, si1, si2, si3), (so0, so1, so2, so3)

    def idx_dma(rb, p):
      return pltpu.async_copy(
          idx_hbm.at[pl.ds(rb * 8, 8), pl.ds(b0, _BC)], idx_bufs[p],
          in_sems[p])

    def out_dma(rb, p):
      return pltpu.async_copy(
          out_bufs[p],
          out_hbm.at[pl.ds(rb * 8, 8), pl.ds(wid * _BT * _D, _BT * _D), :],
          out_sems[p])

    in_h = [None, None, None, None]
    for rb in range(min(3, _RB)):
      in_h[rb & 3] = idx_dma(rb, rb & 3)
    out_h = [None, None, None, None]
    for rb in range(_RB):
      p = rb & 3
      if rb + 3 < _RB:
        in_h[(rb + 3) & 3] = idx_dma(rb + 3, (rb + 3) & 3)
      in_h[p].wait()
      if out_h[p] is not None:
        out_h[p].wait()
      idx_v, out_v = idx_bufs[p], out_bufs[p]

      @plsc.parallel_loop(0, 8 * _BT)
      def body(g):
        j = g >> 2
        bt = g & 3
        for m in range(8):
          iv = idx_v[j, pl.ds(bt * 128 + m * 16, 16)]
          m0 = iv == 0
          m1 = iv == 1
          for c in range(_D):
            out_v[j, bt * _D + c, pl.ds(m * 16, 16)] = jnp.where(
                m0, lut[0][c], jnp.where(m1, lut[1][c], lut[2][c]))

      out_h[p] = out_dma(rb, p)
    for p in range(4):
      if out_h[p] is not None:
        out_h[p].wait()

  return k(idx2, pk)


def kernel(inputs, table, W, b):
  idx2 = inputs.T.astype(jnp.int32)
  pk = jnp.zeros((32,), jnp.float32)
  pk = pk.at[2:8].set(table.reshape(-1).astype(jnp.float32))
  pk = pk.at[8:16].set(W.reshape(-1).astype(jnp.float32))
  pk = pk.at[16:20].set(b.astype(jnp.float32))
  out = _sc_lookup(idx2, pk)
  out = out.reshape(_L, _B // 128, _D, 128)
  return out.transpose(1, 3, 0, 2).reshape(_B, _L, _D)


# speedup vs baseline: 1.4196x; 1.0063x over previous
import functools

import jax
import jax.numpy as jnp
from jax import lax
from jax.experimental import pallas as pl
from jax.experimental.pallas import tpu as pltpu
from jax.experimental.pallas import tpu_sc as plsc

_B = 16384
_L = 200
_D = 4
_NC = 2
_NS = 16
_NW = _NC * _NS
_BC = _B // _NW
_RB = _L // 8
_BT = _BC // 128


def _sc_lookup(idx2, pk):
  mesh = plsc.VectorSubcoreMesh(core_axis_name="c", subcore_axis_name="s")

  @functools.partial(
      pl.kernel,
      mesh=mesh,
      compiler_params=pltpu.CompilerParams(needs_layout_passes=False),
      out_type=jax.ShapeDtypeStruct((_L, _B // 128 * _D, 128), jnp.float32),
      scratch_types=[
          pltpu.VMEM((32,), jnp.float32),
          pltpu.VMEM((8, _BC), jnp.int32),
          pltpu.VMEM((8, _BC), jnp.int32),
          pltpu.VMEM((8, _BC), jnp.int32),
          pltpu.VMEM((8, _BC), jnp.int32),
          pltpu.VMEM((8, _BT * _D, 128), jnp.float32),
          pltpu.VMEM((8, _BT * _D, 128), jnp.float32),
          pltpu.VMEM((8, _BT * _D, 128), jnp.float32),
          pltpu.VMEM((8, _BT * _D, 128), jnp.float32),
          pltpu.SemaphoreType.DMA,
          pltpu.SemaphoreType.DMA,
          pltpu.SemaphoreType.DMA,
          pltpu.SemaphoreType.DMA,
          pltpu.SemaphoreType.DMA,
          pltpu.SemaphoreType.DMA,
          pltpu.SemaphoreType.DMA,
          pltpu.SemaphoreType.DMA,
      ],
  )
  def k(idx_hbm, pk_hbm, out_hbm, pk_v, idx_v0, idx_v1, idx_v2, idx_v3,
        out_v0, out_v1, out_v2, out_v3,
        si0, si1, si2, si3, so0, so1, so2, so3):
    wid = lax.axis_index("s") * _NC + lax.axis_index("c")
    b0 = wid * _BC
    pltpu.sync_copy(pk_hbm, pk_v)
    def splat(i):
      return plsc.load_gather(pk_v, [jnp.full((16,), i, jnp.int32)])

    lut = [[splat(2 + 2 * v) * splat(8 + c) + splat(3 + 2 * v) * splat(12 + c)
            + splat(16 + c) for c in range(_D)] for v in range(3)]

    idx_bufs = (idx_v0, idx_v1, idx_v2, idx_v3)
    out_bufs = (out_v0, out_v1, out_v2, out_v3)
    in_sems, out_sems = (si0, si1, si2, si3), (so0, so1, so2, so3)

    def idx_dma(rb, p):
      return pltpu.async_copy(
          idx_hbm.at[pl.ds(rb * 8, 8), pl.ds(b0, _BC)], idx_bufs[p],
          in_sems[p])

    def out_dma(rb, p):
      return pltpu.async_copy(
          out_bufs[p],
          out_hbm.at[pl.ds(rb * 8, 8), pl.ds(wid * _BT * _D, _BT * _D), :],
          out_sems[p])

    in_h = [None, None, None, None]
    for rb in range(min(3, _RB)):
      in_h[rb & 3] = idx_dma(rb, rb & 3)
    out_h = [None, None, None, None]
    for rb in range(_RB):
      p = rb & 3
      if rb + 3 < _RB:
        in_h[(rb + 3) & 3] = idx_dma(rb + 3, (rb + 3) & 3)
      in_h[p].wait()
      if out_h[p] is not None:
        out_h[p].wait()
      idx_v, out_v = idx_bufs[p], out_bufs[p]

      @plsc.parallel_loop(0, 8 * _BT)
      def body(g):
        j = g >> 2
        bt = g & 3
        for m in range(8):
          iv = idx_v[j, pl.ds(bt * 128 + m * 16, 16)]
          m0 = iv == 0
          m1 = iv == 1
          for c in range(_D):
            out_v[j, bt * _D + c, pl.ds(m * 16, 16)] = jnp.where(
                m0, lut[0][c], jnp.where(m1, lut[1][c], lut[2][c]))

      out_h[p] = out_dma(rb, p)
    for p in range(4):
      if out_h[p] is not None:
        out_h[p].wait()

  return k(idx2, pk)


def kernel(inputs, table, W, b):
  idx2 = inputs.T.astype(jnp.int32)
  pk = jnp.concatenate([
      jnp.zeros((2,), jnp.float32),
      table.reshape(-1).astype(jnp.float32),
      W.reshape(-1).astype(jnp.float32),
      b.astype(jnp.float32),
      jnp.zeros((12,), jnp.float32),
  ])
  out = _sc_lookup(idx2, pk)
  out = out.reshape(_L, _B // 128, _D, 128)
  return out.transpose(1, 3, 0, 2).reshape(_B, _L, _D)
